# baseline (speedup 1.0000x reference)
import jax
import jax.numpy as jnp
from jax.experimental import pallas as pl

_BLK = 2048


def _fused_body(x_ref, aux_ref, wl_ref, o_ref):
    f = x_ref.shape[1]
    hid = wl_ref.shape[0]
    w = aux_ref[0:f, :]
    bv = aux_ref[f:f + 1, :]
    bl = aux_ref[f + 1, 0]
    a = jnp.dot(x_ref[:], w, preferred_element_type=jnp.float32) + bv
    u = jax.nn.sigmoid(a)
    v = u + u - 1.0
    h = jnp.maximum(u * jnp.roll(v, -hid, axis=1), 0.0)
    wl2 = jnp.concatenate(
        [wl_ref[:], jnp.zeros((hid, 1), jnp.float32)], axis=0
    )
    col = jnp.dot(h, wl2, preferred_element_type=jnp.float32) + bl
    o_ref[:] = col.reshape(o_ref.shape)


def kernel(x, edge_index, edge_weight, W_xz, b_xz, W_hz, b_hz, W_xr, b_xr,
           W_hr, b_hr, W_xh, b_xh, W_hh, b_hh, W_lin, b_lin):
    n, f = x.shape
    hid = W_xz.shape[1]
    aux = jnp.concatenate(
        [
            jnp.concatenate([-W_xz, 2.0 * W_xh], axis=1),
            jnp.concatenate(
                [-(b_xz + b_hz), 2.0 * (b_xh + b_hh)]
            ).reshape(1, 2 * hid),
            jnp.pad(b_lin, (0, 2 * hid - 1)).reshape(1, 2 * hid),
            jnp.zeros((6, 2 * hid), jnp.float32),
        ],
        axis=0,
    )
    rows = _BLK // 128
    nsteps = pl.cdiv(n, _BLK)
    out_t = pl.pallas_call(
        _fused_body,
        grid=(nsteps,),
        in_specs=[
            pl.BlockSpec((_BLK, f), lambda i: (i, 0)),
            pl.BlockSpec((f + 8, 2 * hid), lambda i: (0, 0)),
            pl.BlockSpec((hid, 1), lambda i: (0, 0)),
        ],
        out_specs=pl.BlockSpec((rows, 128), lambda i: (i, 0)),
        out_shape=jax.ShapeDtypeStruct((nsteps * rows, 128), jnp.float32),
    )(x, aux, W_lin)
    return out_t.reshape(nsteps * _BLK, 1)[:n]


# speedup vs baseline: 1.0621x; 1.0621x over previous
import functools

import jax
import jax.numpy as jnp
from jax.experimental import pallas as pl
from jax.experimental.pallas import tpu as pltpu

_BLK = 1024


def _fused_body(n, xa_ref, xb_ref, wz_ref, wh_ref, bxz_ref, bhz_ref, bxh_ref,
                bhh_ref, wl_ref, bl_ref, o_ref, wbd_s, bv_s, wl4_s):
    f = xa_ref.shape[1]
    hid = wz_ref.shape[1]
    half = o_ref.shape[0] // 2

    @pl.when(pl.program_id(0) == 0)
    def _build():
        wpk = jnp.concatenate([-wz_ref[:], 2.0 * wh_ref[:]], axis=1)
        zf = jnp.zeros((f, 2 * hid), jnp.float32)
        wbd_s[:] = jnp.concatenate(
            [
                jnp.concatenate([wpk, zf], axis=1),
                jnp.concatenate([zf, wpk], axis=1),
            ],
            axis=0,
        )
        bz = -(bxz_ref[:] + bhz_ref[:])
        bh = 2.0 * (bxh_ref[:] + bhh_ref[:])
        bv_s[:] = jnp.concatenate([bz, bh, bz, bh], axis=1)
        wl = wl_ref[:]
        zc = jnp.zeros((hid, 1), jnp.float32)
        wl4_s[:] = jnp.concatenate(
            [
                jnp.concatenate([wl, zc, zc, zc], axis=0),
                jnp.concatenate([zc, zc, wl, zc], axis=0),
            ],
            axis=1,
        )

    nrows = n - (2 * pl.program_id(0) + 1) * xb_ref.shape[0]
    rmask = (
        jax.lax.broadcasted_iota(jnp.int32, xb_ref.shape, 0) < nrows
    )
    xb = jnp.where(rmask, xb_ref[:], 0.0)
    xab = jnp.concatenate([xa_ref[:], xb], axis=1)
    a = (
        jnp.dot(xab, wbd_s[:], preferred_element_type=jnp.float32)
        + bv_s[:]
    )
    u = jax.nn.sigmoid(a)
    v = u + u - 1.0
    h = jnp.maximum(u * jnp.roll(v, -hid, axis=1), 0.0)
    d = jnp.dot(h, wl4_s[:], preferred_element_type=jnp.float32) + bl_ref[0, 0]
    o_ref[:half, :] = d[:, 0:1].reshape(half, 128)
    o_ref[half:, :] = d[:, 1:2].reshape(half, 128)


def kernel(x, edge_index, edge_weight, W_xz, b_xz, W_hz, b_hz, W_xr, b_xr,
           W_hr, b_hr, W_xh, b_xh, W_hh, b_hh, W_lin, b_lin):
    n, f = x.shape
    hid = W_xz.shape[1]
    rows = 2 * _BLK // 128
    nsteps = pl.cdiv(n, 2 * _BLK)
    _vec = pl.BlockSpec((1, hid), lambda i: (0, 0))
    out_t = pl.pallas_call(
        functools.partial(_fused_body, n),
        grid=(nsteps,),
        in_specs=[
            pl.BlockSpec((_BLK, f), lambda i: (2 * i, 0)),
            pl.BlockSpec((_BLK, f), lambda i: (2 * i + 1, 0)),
            pl.BlockSpec((f, hid), lambda i: (0, 0)),
            pl.BlockSpec((f, hid), lambda i: (0, 0)),
            _vec, _vec, _vec, _vec,
            pl.BlockSpec((hid, 1), lambda i: (0, 0)),
            pl.BlockSpec((1, 1), lambda i: (0, 0)),
        ],
        out_specs=pl.BlockSpec((rows, 128), lambda i: (i, 0)),
        out_shape=jax.ShapeDtypeStruct((nsteps * rows, 128), jnp.float32),
        scratch_shapes=[
            pltpu.VMEM((2 * f, 128), jnp.float32),
            pltpu.VMEM((1, 128), jnp.float32),
            pltpu.VMEM((128, 2), jnp.float32),
        ],
    )(x, x, W_xz, W_xh, b_xz.reshape(1, hid), b_hz.reshape(1, hid),
      b_xh.reshape(1, hid), b_hh.reshape(1, hid), W_lin, b_lin.reshape(1, 1))
    return out_t.reshape(nsteps * 2 * _BLK, 1)[:n]


# speedup vs baseline: 1.0688x; 1.0062x over previous
import functools

import jax
import jax.numpy as jnp
from jax.experimental import pallas as pl
from jax.experimental.pallas import tpu as pltpu

_BLK = 1024


def _fused_body(n, xa_ref, xb_ref, wz_ref, wh_ref, bxz_ref, bhz_ref, bxh_ref,
                bhh_ref, wl_ref, bl_ref, o_ref, wbd_s, bv_s, wl4_s):
    f = xa_ref.shape[1]
    hid = wz_ref.shape[1]
    half = o_ref.shape[0] // 2

    @pl.when(pl.program_id(0) == 0)
    def _build():
        wpk = jnp.concatenate([-0.5 * wz_ref[:], wh_ref[:]], axis=1)
        zf = jnp.zeros((f, 2 * hid), jnp.float32)
        wbd_s[:] = jnp.concatenate(
            [
                jnp.concatenate([wpk, zf], axis=1),
                jnp.concatenate([zf, wpk], axis=1),
            ],
            axis=0,
        )
        bz = -0.5 * (bxz_ref[:] + bhz_ref[:])
        bh = bxh_ref[:] + bhh_ref[:]
        bv_s[:] = jnp.concatenate([bz, bh, bz, bh], axis=1)
        wl = 0.5 * wl_ref[:]
        zc = jnp.zeros((hid, 1), jnp.float32)
        wl4_s[:] = jnp.concatenate(
            [
                jnp.concatenate([wl, zc, zc, zc], axis=0),
                jnp.concatenate([zc, zc, wl, zc], axis=0),
            ],
            axis=1,
        )

    nrows = n - (2 * pl.program_id(0) + 1) * xb_ref.shape[0]
    rmask = (
        jax.lax.broadcasted_iota(jnp.int32, xb_ref.shape, 0) < nrows
    )
    xb = jnp.where(rmask, xb_ref[:], 0.0)
    xab = jnp.concatenate([xa_ref[:], xb], axis=1)
    c = (
        jnp.dot(xab, wbd_s[:], preferred_element_type=jnp.float32)
        + bv_s[:]
    )
    t = jnp.tanh(c)
    h = jnp.maximum((1.0 + t) * jnp.roll(t, -hid, axis=1), 0.0)
    d = jnp.dot(h, wl4_s[:], preferred_element_type=jnp.float32)
    bl = bl_ref[0, 0]
    o_ref[:half, :] = d[:, 0:1].reshape(half, 128) + bl
    o_ref[half:, :] = d[:, 1:2].reshape(half, 128) + bl


def kernel(x, edge_index, edge_weight, W_xz, b_xz, W_hz, b_hz, W_xr, b_xr,
           W_hr, b_hr, W_xh, b_xh, W_hh, b_hh, W_lin, b_lin):
    n, f = x.shape
    hid = W_xz.shape[1]
    rows = 2 * _BLK // 128
    nsteps = pl.cdiv(n, 2 * _BLK)
    _vec = pl.BlockSpec((1, hid), lambda i: (0, 0))
    out_t = pl.pallas_call(
        functools.partial(_fused_body, n),
        grid=(nsteps,),
        in_specs=[
            pl.BlockSpec((_BLK, f), lambda i: (2 * i, 0)),
            pl.BlockSpec((_BLK, f), lambda i: (2 * i + 1, 0)),
            pl.BlockSpec((f, hid), lambda i: (0, 0)),
            pl.BlockSpec((f, hid), lambda i: (0, 0)),
            _vec, _vec, _vec, _vec,
            pl.BlockSpec((hid, 1), lambda i: (0, 0)),
            pl.BlockSpec((1, 1), lambda i: (0, 0)),
        ],
        out_specs=pl.BlockSpec((rows, 128), lambda i: (i, 0)),
        out_shape=jax.ShapeDtypeStruct((nsteps * rows, 128), jnp.float32),
        scratch_shapes=[
            pltpu.VMEM((2 * f, 128), jnp.float32),
            pltpu.VMEM((1, 128), jnp.float32),
            pltpu.VMEM((128, 2), jnp.float32),
        ],
    )(x, x, W_xz, W_xh, b_xz.reshape(1, hid), b_hz.reshape(1, hid),
      b_xh.reshape(1, hid), b_hh.reshape(1, hid), W_lin, b_lin.reshape(1, 1))
    return out_t.reshape(nsteps * 2 * _BLK, 1)[:n]


# speedup vs baseline: 1.0700x; 1.0011x over previous
import functools

import jax
import jax.numpy as jnp
from jax.experimental import pallas as pl
from jax.experimental.pallas import tpu as pltpu

_BLK = 1024


def _fused_body(n, x_ref, aux_ref, o_ref, wbd_s, bv_s, wp_s):
    blk, f = x_ref.shape
    blk = blk // 2
    hid = aux_ref.shape[1]
    half = o_ref.shape[0] // 2

    @pl.when(pl.program_id(0) == 0)
    def _build():
        wpk = jnp.concatenate(
            [-0.5 * aux_ref[0:f, :], aux_ref[f:2 * f, :]], axis=1
        )
        zf = jnp.zeros((f, 2 * hid), jnp.float32)
        wbd_s[:] = jnp.concatenate(
            [
                jnp.concatenate([wpk, zf], axis=1),
                jnp.concatenate([zf, wpk], axis=1),
            ],
            axis=0,
        )
        bz = -0.5 * (aux_ref[2 * f:2 * f + 1, :] + aux_ref[2 * f + 1:2 * f + 2, :])
        bh = aux_ref[2 * f + 2:2 * f + 3, :] + aux_ref[2 * f + 3:2 * f + 4, :]
        bv_s[:] = jnp.concatenate([bz, bh, bz, bh], axis=1)
        wlr = 0.5 * aux_ref[2 * f + 4:2 * f + 5, :]
        z1 = jnp.zeros((1, hid), jnp.float32)
        wp_s[0:1, :] = jnp.concatenate([wlr, z1, z1, z1], axis=1)
        wp_s[1:2, :] = jnp.concatenate([z1, z1, wlr, z1], axis=1)

    base_b = 2 * pl.program_id(0) * blk + blk
    rmask = (
        jax.lax.broadcasted_iota(jnp.int32, (blk, f), 0) < (n - base_b)
    )
    xb = jnp.where(rmask, x_ref[blk:, :], 0.0)
    xab = jnp.concatenate([x_ref[:blk, :], xb], axis=1)
    c = (
        jnp.dot(xab, wbd_s[:], preferred_element_type=jnp.float32)
        + bv_s[:]
    )
    t = jnp.tanh(c)
    h = jnp.maximum((1.0 + t) * jnp.roll(t, -hid, axis=1), 0.0)
    d = jax.lax.dot_general(
        h, wp_s[0:2, :], (((1,), (1,)), ((), ())),
        preferred_element_type=jnp.float32,
    )
    bl = aux_ref[2 * f + 5, 0]
    o_ref[:half, :] = d[:, 0:1].reshape(half, 128) + bl
    o_ref[half:, :] = d[:, 1:2].reshape(half, 128) + bl


def kernel(x, edge_index, edge_weight, W_xz, b_xz, W_hz, b_hz, W_xr, b_xr,
           W_hr, b_hr, W_xh, b_xh, W_hh, b_hh, W_lin, b_lin):
    n, f = x.shape
    hid = W_xz.shape[1]
    aux = jnp.concatenate(
        [
            W_xz,
            W_xh,
            b_xz[None, :], b_hz[None, :],
            b_xh[None, :], b_hh[None, :],
            W_lin.reshape(1, hid),
            jnp.broadcast_to(b_lin, (hid,))[None, :],
            jnp.zeros((2, hid), jnp.float32),
        ],
        axis=0,
    )
    rows = 2 * _BLK // 128
    nsteps = pl.cdiv(n, 2 * _BLK)
    out_t = pl.pallas_call(
        functools.partial(_fused_body, n),
        grid=(nsteps,),
        in_specs=[
            pl.BlockSpec((2 * _BLK, f), lambda i: (i, 0)),
            pl.BlockSpec((2 * f + 8, hid), lambda i: (0, 0)),
        ],
        out_specs=pl.BlockSpec((rows, 128), lambda i: (i, 0)),
        out_shape=jax.ShapeDtypeStruct((nsteps * rows, 128), jnp.float32),
        scratch_shapes=[
            pltpu.VMEM((2 * f, 128), jnp.float32),
            pltpu.VMEM((1, 128), jnp.float32),
            pltpu.VMEM((8, 128), jnp.float32),
        ],
    )(x, aux)
    return out_t.reshape(nsteps * 2 * _BLK, 1)[:n]


# speedup vs baseline: 1.1760x; 1.0991x over previous
import functools

import jax
import jax.numpy as jnp
from jax.experimental import pallas as pl
from jax.experimental.pallas import tpu as pltpu

_BLK = 5120


def _fused_body(n, x_ref, aux_ref, o_ref, wbd_s, bv_s, wp_s):
    blk, f = x_ref.shape
    blk = blk // 2
    hid = aux_ref.shape[1]
    half = o_ref.shape[0] // 2

    @pl.when(pl.program_id(0) == 0)
    def _build():
        wpk = jnp.concatenate(
            [-0.5 * aux_ref[0:f, :], aux_ref[f:2 * f, :]], axis=1
        )
        zf = jnp.zeros((f, 2 * hid), jnp.float32)
        wbd_s[:] = jnp.concatenate(
            [
                jnp.concatenate([wpk, zf], axis=1),
                jnp.concatenate([zf, wpk], axis=1),
            ],
            axis=0,
        )
        bz = -0.5 * (aux_ref[2 * f:2 * f + 1, :] + aux_ref[2 * f + 1:2 * f + 2, :])
        bh = aux_ref[2 * f + 2:2 * f + 3, :] + aux_ref[2 * f + 3:2 * f + 4, :]
        bv_s[:] = jnp.concatenate([bz, bh, bz, bh], axis=1)
        wlr = 0.5 * aux_ref[2 * f + 4:2 * f + 5, :]
        z1 = jnp.zeros((1, hid), jnp.float32)
        wp_s[0:1, :] = jnp.concatenate([wlr, z1, z1, z1], axis=1)
        wp_s[1:2, :] = jnp.concatenate([z1, z1, wlr, z1], axis=1)

    base_b = 2 * pl.program_id(0) * blk + blk
    rmask = (
        jax.lax.broadcasted_iota(jnp.int32, (blk, f), 0) < (n - base_b)
    )
    xb = jnp.where(rmask, x_ref[blk:, :], 0.0)
    xab = jnp.concatenate([x_ref[:blk, :], xb], axis=1)
    c = (
        jnp.dot(xab, wbd_s[:], preferred_element_type=jnp.float32)
        + bv_s[:]
    )
    t = jnp.tanh(c)
    h = jnp.maximum((1.0 + t) * jnp.roll(t, -hid, axis=1), 0.0)
    d = jax.lax.dot_general(
        h, wp_s[0:2, :], (((1,), (1,)), ((), ())),
        preferred_element_type=jnp.float32,
    )
    bl = aux_ref[2 * f + 5, 0]
    o_ref[:half, :] = d[:, 0:1].reshape(half, 128) + bl
    o_ref[half:, :] = d[:, 1:2].reshape(half, 128) + bl


def kernel(x, edge_index, edge_weight, W_xz, b_xz, W_hz, b_hz, W_xr, b_xr,
           W_hr, b_hr, W_xh, b_xh, W_hh, b_hh, W_lin, b_lin):
    n, f = x.shape
    hid = W_xz.shape[1]
    aux = jnp.concatenate(
        [
            W_xz,
            W_xh,
            b_xz[None, :], b_hz[None, :],
            b_xh[None, :], b_hh[None, :],
            W_lin.reshape(1, hid),
            jnp.broadcast_to(b_lin, (hid,))[None, :],
            jnp.zeros((2, hid), jnp.float32),
        ],
        axis=0,
    )
    rows = 2 * _BLK // 128
    nsteps = pl.cdiv(n, 2 * _BLK)
    out_t = pl.pallas_call(
        functools.partial(_fused_body, n),
        grid=(nsteps,),
        in_specs=[
            pl.BlockSpec((2 * _BLK, f), lambda i: (i, 0)),
            pl.BlockSpec((2 * f + 8, hid), lambda i: (0, 0)),
        ],
        out_specs=pl.BlockSpec((rows, 128), lambda i: (i, 0)),
        out_shape=jax.ShapeDtypeStruct((nsteps * rows, 128), jnp.float32),
        scratch_shapes=[
            pltpu.VMEM((2 * f, 128), jnp.float32),
            pltpu.VMEM((1, 128), jnp.float32),
            pltpu.VMEM((8, 128), jnp.float32),
        ],
    )(x, aux)
    return out_t.reshape(nsteps * 2 * _BLK, 1)[:n]
